# baseline (speedup 1.0000x reference)
import functools

import jax
import jax.numpy as jnp
from jax import lax
from jax.experimental import pallas as pl
from jax.experimental.pallas import tpu as pltpu
from jax.experimental.pallas import tpu_sc as plsc


def _encode_body(x_ref, feW1_ref, feb1_ref, feW2_ref, feb2_ref, projW_ref,
                 projb_ref, cb_ref, idx_ref, *, n_cat, levels, feats, k_codes):
    f32 = jnp.float32
    x = x_ref[...]
    blk = x.shape[0]
    ones_col = jnp.ones((blk, 1), f32)
    for c in range(n_cat):
        xc = x[:, c * feats:(c + 1) * feats]
        h = jnp.dot(xc, feW1_ref[c], preferred_element_type=f32)
        h = jnp.maximum(h + feb1_ref[c:c + 1, :], 0.0)
        emb = jnp.dot(h, feW2_ref[c], preferred_element_type=f32)
        emb = emb + feb2_ref[c:c + 1, :]
        for l in range(levels):
            z = jnp.dot(emb, projW_ref[c, l], preferred_element_type=f32)
            z = z + projb_ref[c, l:l + 1, :]
            cb = cb_ref[c, l]
            cross = lax.dot_general(
                z, cb, (((1,), (1,)), ((), ())),
                preferred_element_type=f32)
            z2 = jnp.sum(z * z, axis=-1, keepdims=True)
            e2 = jnp.sum(cb * cb, axis=-1)
            dist = z2 - 2.0 * cross + e2[None, :]
            idx = jnp.argmin(dist, axis=-1).astype(jnp.int32)
            j = c * levels + l
            idx_ref[:, j] = idx + j * k_codes


def _encode(x, fe_W1, fe_b1, fe_W2, fe_b2, proj_W, proj_b, codebooks, blk):
    bsz, in_dim = x.shape
    n_cat, feats, _ = fe_W1.shape
    levels, k_codes = codebooks.shape[1], codebooks.shape[2]
    grid = (bsz // blk,)

    def rep(shape):
        return pl.BlockSpec(shape, lambda i: (0,) * len(shape))

    body = functools.partial(_encode_body, n_cat=n_cat, levels=levels,
                             feats=feats, k_codes=k_codes)
    return pl.pallas_call(
        body,
        grid=grid,
        in_specs=[
            pl.BlockSpec((blk, in_dim), lambda i: (i, 0)),
            rep(fe_W1.shape), rep(fe_b1.shape),
            rep(fe_W2.shape), rep(fe_b2.shape),
            rep(proj_W.shape), rep(proj_b.shape),
            rep(codebooks.shape),
        ],
        out_specs=pl.BlockSpec((blk, n_cat * levels), lambda i: (i, 0)),
        out_shape=jax.ShapeDtypeStruct((bsz, n_cat * levels), jnp.int32),
    )(x, fe_W1, fe_b1, fe_W2, fe_b2, proj_W, proj_b, codebooks)


def _sc_gather(table, idx_flat, d):
    n = idx_flat.shape[0]
    info = plsc.get_sparse_core_info()
    nc, ns = info.num_cores, info.num_subcores
    nw = nc * ns
    chunk = 128
    n_chunks = n // (nw * chunk)
    per_w = n_chunks * chunk
    idx2d = idx_flat.reshape(n // chunk, chunk)
    mesh = plsc.VectorSubcoreMesh(core_axis_name="c", subcore_axis_name="s")

    @functools.partial(
        pl.kernel, mesh=mesh,
        compiler_params=pltpu.CompilerParams(use_tc_tiling_on_sc=False),
        out_type=jax.ShapeDtypeStruct((n, d), jnp.float32),
        scratch_types=[
            pltpu.VMEM((n_chunks, chunk), jnp.int32),
            pltpu.VMEM((per_w, d), jnp.float32),
            pltpu.SemaphoreType.DMA,
        ],
    )
    def gather_k(table_hbm, idx_hbm, out_hbm, idx_v, rows_v, sem):
        wid = lax.axis_index("s") * nc + lax.axis_index("c")
        pltpu.sync_copy(idx_hbm.at[pl.ds(wid * n_chunks, n_chunks)], idx_v)
        copies = [
            pltpu.async_copy(table_hbm.at[idx_v.at[j]],
                             rows_v.at[pl.ds(j * chunk, chunk)], sem)
            for j in range(n_chunks)
        ]
        for cp in copies:
            cp.wait()
        pltpu.sync_copy(rows_v, out_hbm.at[pl.ds(wid * per_w, per_w)])

    return gather_k(table, idx2d)


def _decode_body(q_ref, decW1_ref, decb1_ref, decW2_ref, decb2_ref, out_ref):
    f32 = jnp.float32
    h2 = jnp.dot(q_ref[...], decW1_ref[...], preferred_element_type=f32)
    h2 = jnp.maximum(h2 + decb1_ref[...], 0.0)
    out = jnp.dot(h2, decW2_ref[...], preferred_element_type=f32)
    out_ref[...] = out + decb2_ref[...]


def _decode(q_flat, dec_W1, dec_b1, dec_W2, dec_b2, blk):
    bsz, flat_d = q_flat.shape
    out_d = dec_W2.shape[1]
    grid = (bsz // blk,)

    def rep(shape):
        return pl.BlockSpec(shape, lambda i: (0,) * len(shape))

    return pl.pallas_call(
        _decode_body,
        grid=grid,
        in_specs=[
            pl.BlockSpec((blk, flat_d), lambda i: (i, 0)),
            rep(dec_W1.shape), rep((1, dec_b1.shape[0])),
            rep(dec_W2.shape), rep((1, dec_b2.shape[0])),
        ],
        out_specs=pl.BlockSpec((blk, out_d), lambda i: (i, 0)),
        out_shape=jax.ShapeDtypeStruct((bsz, out_d), jnp.float32),
    )(q_flat, dec_W1, dec_b1.reshape(1, -1), dec_W2, dec_b2.reshape(1, -1))


def kernel(x, fe_W1, fe_b1, fe_W2, fe_b2, proj_W, proj_b, codebooks,
           dec_W1, dec_b1, dec_W2, dec_b2):
    bsz = x.shape[0]
    n_cat, levels, k_codes, d = codebooks.shape
    table = codebooks.reshape(n_cat * levels * k_codes, d)
    n_split = 2
    chunk_b = bsz // n_split
    outs = []
    for s in range(n_split):
        xs = lax.slice_in_dim(x, s * chunk_b, (s + 1) * chunk_b, axis=0)
        idx = _encode(xs, fe_W1, fe_b1, fe_W2, fe_b2, proj_W, proj_b,
                      codebooks, blk=512)
        q = _sc_gather(table, idx.reshape(chunk_b * n_cat * levels), d)
        q_flat = q.reshape(chunk_b, n_cat * levels * d)
        outs.append(_decode(q_flat, dec_W1, dec_b1, dec_W2, dec_b2, blk=512))
    return jnp.concatenate(outs, axis=0)


# speedup vs baseline: 1.2879x; 1.2879x over previous
import functools

import jax
import jax.numpy as jnp
from jax import lax
from jax.experimental import pallas as pl
from jax.experimental.pallas import tpu as pltpu
from jax.experimental.pallas import tpu_sc as plsc


def _encode_body(x_ref, feW1_ref, feb1_ref, feW2_ref, feb2_ref, projW_ref,
                 projb_ref, cb_ref, idx_ref, cba_ref, e2_ref, *,
                 n_cat, levels, feats, k_codes):
    f32 = jnp.float32

    @pl.when(pl.program_id(0) == 0)
    def _init():
        for c in range(n_cat):
            for l in range(levels):
                j = c * levels + l
                cb = cb_ref[c, l]
                cba_ref[j] = jnp.concatenate(
                    [jnp.ones_like(cb), cb], axis=1)
                e2_ref[j, :] = jnp.sum(cb * cb, axis=-1)

    x = x_ref[...]
    for c in range(n_cat):
        xc = x[:, c * feats:(c + 1) * feats]
        h = jnp.dot(xc, feW1_ref[c], preferred_element_type=f32)
        h = jnp.maximum(h + feb1_ref[c:c + 1, :], 0.0)
        emb = jnp.dot(h, feW2_ref[c], preferred_element_type=f32)
        emb = emb + feb2_ref[c:c + 1, :]
        for l in range(levels):
            z = jnp.dot(emb, projW_ref[c, l], preferred_element_type=f32)
            z = z + projb_ref[c, l:l + 1, :]
            j = c * levels + l
            za = jnp.concatenate([z * z, -2.0 * z], axis=1)
            dist = lax.dot_general(
                za, cba_ref[j], (((1,), (1,)), ((), ())),
                preferred_element_type=f32)
            dist = dist + e2_ref[j, :][None, :]
            idx = jnp.argmin(dist, axis=-1).astype(jnp.int32)
            idx_ref[:, j] = idx + j * k_codes


def _encode(x, fe_W1, fe_b1, fe_W2, fe_b2, proj_W, proj_b, codebooks, blk):
    bsz, in_dim = x.shape
    n_cat, feats, _ = fe_W1.shape
    levels, k_codes = codebooks.shape[1], codebooks.shape[2]
    grid = (bsz // blk,)

    def rep(shape):
        return pl.BlockSpec(shape, lambda i: (0,) * len(shape))

    body = functools.partial(_encode_body, n_cat=n_cat, levels=levels,
                             feats=feats, k_codes=k_codes)
    return pl.pallas_call(
        body,
        grid=grid,
        in_specs=[
            pl.BlockSpec((blk, in_dim), lambda i: (i, 0)),
            rep(fe_W1.shape), rep(fe_b1.shape),
            rep(fe_W2.shape), rep(fe_b2.shape),
            rep(proj_W.shape), rep(proj_b.shape),
            rep(codebooks.shape),
        ],
        out_specs=pl.BlockSpec((blk, n_cat * levels), lambda i: (i, 0)),
        out_shape=jax.ShapeDtypeStruct((bsz, n_cat * levels), jnp.int32),
        scratch_shapes=[
            pltpu.VMEM((n_cat * levels, k_codes, 2 * codebooks.shape[3]),
                       jnp.float32),
            pltpu.VMEM((n_cat * levels, k_codes), jnp.float32),
        ],
    )(x, fe_W1, fe_b1, fe_W2, fe_b2, proj_W, proj_b, codebooks)


def _sc_gather(table, idx_flat, d):
    n = idx_flat.shape[0]
    info = plsc.get_sparse_core_info()
    nc, ns = info.num_cores, info.num_subcores
    nw = nc * ns
    chunk = 128
    n_chunks = n // (nw * chunk)
    per_w = n_chunks * chunk
    idx2d = idx_flat.reshape(n // chunk, chunk)
    mesh = plsc.VectorSubcoreMesh(core_axis_name="c", subcore_axis_name="s")

    @functools.partial(
        pl.kernel, mesh=mesh,
        compiler_params=pltpu.CompilerParams(use_tc_tiling_on_sc=False),
        out_type=jax.ShapeDtypeStruct((n, d), jnp.float32),
        scratch_types=[
            pltpu.VMEM((n_chunks, chunk), jnp.int32),
            pltpu.VMEM((per_w, d), jnp.float32),
            pltpu.SemaphoreType.DMA,
        ],
    )
    def gather_k(table_hbm, idx_hbm, out_hbm, idx_v, rows_v, sem):
        wid = lax.axis_index("s") * nc + lax.axis_index("c")
        pltpu.sync_copy(idx_hbm.at[pl.ds(wid * n_chunks, n_chunks)], idx_v)
        copies = [
            pltpu.async_copy(table_hbm.at[idx_v.at[j]],
                             rows_v.at[pl.ds(j * chunk, chunk)], sem)
            for j in range(n_chunks)
        ]
        for cp in copies:
            cp.wait()
        pltpu.sync_copy(rows_v, out_hbm.at[pl.ds(wid * per_w, per_w)])

    return gather_k(table, idx2d)


def _decode_body(q_ref, decW1_ref, decb1_ref, decW2_ref, decb2_ref, out_ref):
    f32 = jnp.float32
    h2 = jnp.dot(q_ref[...], decW1_ref[...], preferred_element_type=f32)
    h2 = jnp.maximum(h2 + decb1_ref[...], 0.0)
    out = jnp.dot(h2, decW2_ref[...], preferred_element_type=f32)
    out_ref[...] = out + decb2_ref[...]


def _decode(q_flat, dec_W1, dec_b1, dec_W2, dec_b2, blk):
    bsz, flat_d = q_flat.shape
    out_d = dec_W2.shape[1]
    grid = (bsz // blk,)

    def rep(shape):
        return pl.BlockSpec(shape, lambda i: (0,) * len(shape))

    return pl.pallas_call(
        _decode_body,
        grid=grid,
        in_specs=[
            pl.BlockSpec((blk, flat_d), lambda i: (i, 0)),
            rep(dec_W1.shape), rep((1, dec_b1.shape[0])),
            rep(dec_W2.shape), rep((1, dec_b2.shape[0])),
        ],
        out_specs=pl.BlockSpec((blk, out_d), lambda i: (i, 0)),
        out_shape=jax.ShapeDtypeStruct((bsz, out_d), jnp.float32),
    )(q_flat, dec_W1, dec_b1.reshape(1, -1), dec_W2, dec_b2.reshape(1, -1))


def kernel(x, fe_W1, fe_b1, fe_W2, fe_b2, proj_W, proj_b, codebooks,
           dec_W1, dec_b1, dec_W2, dec_b2):
    bsz = x.shape[0]
    n_cat, levels, k_codes, d = codebooks.shape
    table = codebooks.reshape(n_cat * levels * k_codes, d)
    n_split = 2
    chunk_b = bsz // n_split
    outs = []
    for s in range(n_split):
        xs = lax.slice_in_dim(x, s * chunk_b, (s + 1) * chunk_b, axis=0)
        idx = _encode(xs, fe_W1, fe_b1, fe_W2, fe_b2, proj_W, proj_b,
                      codebooks, blk=512)
        q = _sc_gather(table, idx.reshape(chunk_b * n_cat * levels), d)
        q_flat = q.reshape(chunk_b, n_cat * levels * d)
        outs.append(_decode(q_flat, dec_W1, dec_b1, dec_W2, dec_b2, blk=512))
    return jnp.concatenate(outs, axis=0)
